# Initial kernel scaffold; baseline (speedup 1.0000x reference)
import jax
import jax.numpy as jnp
from jax.experimental import pallas as pl


def kernel(x, edge_index, W1, b1, W2, b2):
    raise NotImplementedError("write your pallas kernel here")


# speedup vs baseline: 35.4846x; 35.4846x over previous
import functools

import jax
import jax.numpy as jnp
from jax import lax
from jax.experimental import pallas as pl
from jax.experimental.pallas import tpu as pltpu
from jax.experimental.pallas import tpu_sc as plsc

_NC = 2
_NS = 16
_W = _NC * _NS
_LANES = 128
_GK = 8


def _sc_mesh():
    return plsc.VectorSubcoreMesh(core_axis_name="c", subcore_axis_name="s")


def _make_agg(NP, RPT, DH):
    RPS = NP // _NS
    GRP = RPT // _GK

    @functools.partial(
        pl.kernel,
        out_type=jax.ShapeDtypeStruct((_NC * NP, DH), jnp.float32),
        mesh=_sc_mesh(),
        compiler_params=pltpu.CompilerParams(use_tc_tiling_on_sc=False),
        scratch_types=[
            pltpu.VMEM((RPT, _LANES), jnp.int32),
            pltpu.VMEM((RPT, _LANES), jnp.int32),
            pltpu.VMEM((_GK, _LANES, DH), jnp.float32),
            pltpu.VMEM((RPS, DH), jnp.float32),
            pltpu.VMEM_SHARED((NP, DH), jnp.float32),
            pltpu.SemaphoreType.DMA,
            pltpu.SemaphoreType.DMA,
        ],
    )
    def agg(hs, srci, dsti, zrows, out, srcv, dstv, gbuf, stage, acc, gsem, ssem):
        cid = lax.axis_index("c")
        sid = lax.axis_index("s")
        wid = cid * _NS + sid
        pltpu.sync_copy(zrows, stage)
        pltpu.sync_copy(stage, acc.at[pl.ds(sid * RPS, RPS)])
        plsc.subcore_barrier()
        pltpu.sync_copy(srci.at[pl.ds(wid * RPT, RPT)], srcv)
        pltpu.sync_copy(dsti.at[pl.ds(wid * RPT, RPT)], dstv)

        def body(g, carry):
            r0 = g * _GK
            gd = [
                pltpu.async_copy(hs.at[srcv.at[r0 + b]], gbuf.at[b], gsem)
                for b in range(_GK)
            ]
            for d in gd:
                d.wait()
            sd = [
                pltpu.async_copy(gbuf.at[b], acc.at[dstv.at[r0 + b]], ssem,
                                 add=True)
                for b in range(_GK)
            ]
            for d in sd:
                d.wait()
            return carry

        lax.fori_loop(0, GRP, body, 0)
        plsc.subcore_barrier()
        pltpu.sync_copy(acc.at[pl.ds(sid * RPS, RPS)], stage)
        pltpu.sync_copy(stage, out.at[pl.ds(cid * NP + sid * RPS, RPS)])

    return agg


def _make_deg(NP, RPT):
    RPS = NP // _NS
    GRP = RPT // _GK

    @functools.partial(
        pl.kernel,
        out_type=jax.ShapeDtypeStruct((_NC * NP,), jnp.float32),
        mesh=_sc_mesh(),
        compiler_params=pltpu.CompilerParams(use_tc_tiling_on_sc=False),
        scratch_types=[
            pltpu.VMEM((RPT, _LANES), jnp.int32),
            pltpu.VMEM((_LANES,), jnp.float32),
            pltpu.VMEM((RPS,), jnp.float32),
            pltpu.VMEM_SHARED((NP,), jnp.float32),
            pltpu.SemaphoreType.DMA,
        ],
    )
    def deg(dsti, ones_h, zcol, out, dstv, onesv, stage, dsh, sem):
        cid = lax.axis_index("c")
        sid = lax.axis_index("s")
        wid = cid * _NS + sid
        pltpu.sync_copy(zcol, stage)
        pltpu.sync_copy(stage, dsh.at[pl.ds(sid * RPS, RPS)])
        plsc.subcore_barrier()
        pltpu.sync_copy(ones_h, onesv)
        pltpu.sync_copy(dsti.at[pl.ds(wid * RPT, RPT)], dstv)

        def body(g, carry):
            r0 = g * _GK
            sd = [
                pltpu.async_copy(onesv, dsh.at[dstv.at[r0 + b]], sem, add=True)
                for b in range(_GK)
            ]
            for d in sd:
                d.wait()
            return carry

        lax.fori_loop(0, GRP, body, 0)
        plsc.subcore_barrier()
        pltpu.sync_copy(dsh.at[pl.ds(sid * RPS, RPS)], stage)
        pltpu.sync_copy(stage, out.at[pl.ds(cid * NP + sid * RPS, RPS)])

    return deg


def _tc1(degt, xp, W1, BM=1024):
    NP, DI = xp.shape
    DH = W1.shape[1]

    def body(deg_ref, x_ref, w_ref, o_ref):
        dg = deg_ref[:, 0:1] + deg_ref[:, 1:2] + 1.0
        dinv = lax.rsqrt(jnp.maximum(dg, 1e-12))
        h = jnp.dot(x_ref[...], w_ref[...], preferred_element_type=jnp.float32)
        o_ref[...] = h * dinv

    return pl.pallas_call(
        body,
        grid=(NP // BM,),
        in_specs=[
            pl.BlockSpec((BM, 2), lambda i: (i, 0)),
            pl.BlockSpec((BM, DI), lambda i: (i, 0)),
            pl.BlockSpec((DI, DH), lambda i: (0, 0)),
        ],
        out_specs=pl.BlockSpec((BM, DH), lambda i: (i, 0)),
        out_shape=jax.ShapeDtypeStruct((NP, DH), jnp.float32),
    )(degt, xp, W1)


def _tc2(degt, hs1, acc1, b1r, W2p, BM=1024):
    NP, DH = hs1.shape

    def body(deg_ref, hs_ref, acc_ref, b_ref, w_ref, o_ref):
        dg = deg_ref[:, 0:1] + deg_ref[:, 1:2] + 1.0
        dinv = lax.rsqrt(jnp.maximum(dg, 1e-12))
        s = hs_ref[...] + acc_ref[0] + acc_ref[1]
        g = jnp.maximum(dinv * s + b_ref[...], 0.0)
        o_ref[...] = jnp.dot(g, w_ref[...],
                             preferred_element_type=jnp.float32) * dinv

    return pl.pallas_call(
        body,
        grid=(NP // BM,),
        in_specs=[
            pl.BlockSpec((BM, 2), lambda i: (i, 0)),
            pl.BlockSpec((BM, DH), lambda i: (i, 0)),
            pl.BlockSpec((2, BM, DH), lambda i: (0, i, 0)),
            pl.BlockSpec((1, DH), lambda i: (0, 0)),
            pl.BlockSpec((DH, DH), lambda i: (0, 0)),
        ],
        out_specs=pl.BlockSpec((BM, DH), lambda i: (i, 0)),
        out_shape=jax.ShapeDtypeStruct((NP, DH), jnp.float32),
    )(degt, hs1, acc1, b1r, W2p)


def _tc3(degt, hs2, acc2, b2r, BM=1024):
    NP, DH = hs2.shape

    def body(deg_ref, hs_ref, acc_ref, b_ref, o_ref):
        dg = deg_ref[:, 0:1] + deg_ref[:, 1:2] + 1.0
        dinv = lax.rsqrt(jnp.maximum(dg, 1e-12))
        s = hs_ref[...] + acc_ref[0] + acc_ref[1]
        o_ref[...] = dinv * s + b_ref[...]

    return pl.pallas_call(
        body,
        grid=(NP // BM,),
        in_specs=[
            pl.BlockSpec((BM, 2), lambda i: (i, 0)),
            pl.BlockSpec((BM, DH), lambda i: (i, 0)),
            pl.BlockSpec((2, BM, DH), lambda i: (0, i, 0)),
            pl.BlockSpec((1, DH), lambda i: (0, 0)),
        ],
        out_specs=pl.BlockSpec((BM, DH), lambda i: (i, 0)),
        out_shape=jax.ShapeDtypeStruct((NP, DH), jnp.float32),
    )(degt, hs2, acc2, b2r)


def kernel(x, edge_index, W1, b1, W2, b2):
    N, DI = x.shape
    DH = W1.shape[1]
    DO = W2.shape[1]
    E = edge_index.shape[1]

    NP = -(-(N + 1) // 1024) * 1024
    RPT = -(-E // (_LANES * _W * _GK)) * _GK
    EP = _W * RPT * _LANES
    RPS = NP // _NS

    src = edge_index[0]
    dst = edge_index[1]
    pad = jnp.full((EP - E,), N, dtype=jnp.int32)
    srcr = jnp.concatenate([src, pad]).reshape(-1, _LANES)
    dstr = jnp.concatenate([dst, pad]).reshape(-1, _LANES)

    xp = jnp.pad(x, ((0, NP - N), (0, 0)))
    ones = jnp.ones((_LANES,), jnp.float32)
    zcol = jnp.zeros((RPS,), jnp.float32)
    zrows = jnp.zeros((RPS, DH), jnp.float32)
    W2p = jnp.pad(W2, ((0, 0), (0, DH - DO)))
    b1r = b1.reshape(1, DH)
    b2r = jnp.pad(b2, (0, DH - DO)).reshape(1, DH)

    deg2 = _make_deg(NP, RPT)(dstr, ones, zcol).reshape(_NC, NP)
    degt = deg2.T

    agg = _make_agg(NP, RPT, DH)

    hs1 = _tc1(degt, xp, W1)
    acc1 = agg(hs1, srcr, dstr, zrows).reshape(_NC, NP, DH)
    hs2 = _tc2(degt, hs1, acc1, b1r, W2p)
    acc2 = agg(hs2, srcr, dstr, zrows).reshape(_NC, NP, DH)
    outp = _tc3(degt, hs2, acc2, b2r)
    return outp[:N, :DO]


# speedup vs baseline: 52.6446x; 1.4836x over previous
import functools

import jax
import jax.numpy as jnp
from jax import lax
from jax.experimental import pallas as pl
from jax.experimental.pallas import tpu as pltpu
from jax.experimental.pallas import tpu_sc as plsc

_NC = 2
_NS = 16
_W = _NC * _NS
_LANES = 128
_GK = 8


def _sc_mesh():
    return plsc.VectorSubcoreMesh(core_axis_name="c", subcore_axis_name="s")


def _make_agg(NP, RPT, DH):
    RPS = NP // _NS
    GRP = RPT // _GK

    @functools.partial(
        pl.kernel,
        out_type=jax.ShapeDtypeStruct((_NC * NP, DH), jnp.float32),
        mesh=_sc_mesh(),
        compiler_params=pltpu.CompilerParams(use_tc_tiling_on_sc=False),
        scratch_types=[
            pltpu.VMEM((RPT, _LANES), jnp.int32),
            pltpu.VMEM((RPT, _LANES), jnp.int32),
            pltpu.VMEM((_GK, _LANES, DH), jnp.float32),
            pltpu.VMEM((RPS, DH), jnp.float32),
            pltpu.VMEM_SHARED((NP, DH), jnp.float32),
            pltpu.SemaphoreType.DMA,
            pltpu.SemaphoreType.DMA,
        ],
    )
    def agg(hs, srci, dsti, zrows, out, srcv, dstv, gbuf, stage, acc, gsem, ssem):
        cid = lax.axis_index("c")
        sid = lax.axis_index("s")
        wid = cid * _NS + sid
        pltpu.sync_copy(zrows, stage)
        pltpu.sync_copy(stage, acc.at[pl.ds(sid * RPS, RPS)])
        plsc.subcore_barrier()
        pltpu.sync_copy(srci.at[pl.ds(wid * RPT, RPT)], srcv)
        pltpu.sync_copy(dsti.at[pl.ds(wid * RPT, RPT)], dstv)

        def body(g, carry):
            r0 = g * _GK
            gd = [
                pltpu.async_copy(hs.at[srcv.at[r0 + b]], gbuf.at[b], gsem)
                for b in range(_GK)
            ]
            for d in gd:
                d.wait()
            sd = [
                pltpu.async_copy(gbuf.at[b], acc.at[dstv.at[r0 + b]], ssem,
                                 add=True)
                for b in range(_GK)
            ]
            for d in sd:
                d.wait()
            return carry

        lax.fori_loop(0, GRP, body, 0)
        plsc.subcore_barrier()
        pltpu.sync_copy(acc.at[pl.ds(sid * RPS, RPS)], stage)
        pltpu.sync_copy(stage, out.at[pl.ds(cid * NP + sid * RPS, RPS)])

    return agg


def _make_deg(NP, RPT):
    RPS = NP // _NS
    GRP = RPT // _GK

    @functools.partial(
        pl.kernel,
        out_type=jax.ShapeDtypeStruct((_NC * NP,), jnp.float32),
        mesh=_sc_mesh(),
        compiler_params=pltpu.CompilerParams(use_tc_tiling_on_sc=False),
        scratch_types=[
            pltpu.VMEM((RPT, _LANES), jnp.int32),
            pltpu.VMEM((_LANES,), jnp.float32),
            pltpu.VMEM((RPS,), jnp.float32),
            pltpu.VMEM_SHARED((NP,), jnp.float32),
            pltpu.SemaphoreType.DMA,
        ],
    )
    def deg(dsti, ones_h, zcol, out, dstv, onesv, stage, dsh, sem):
        cid = lax.axis_index("c")
        sid = lax.axis_index("s")
        wid = cid * _NS + sid
        pltpu.sync_copy(zcol, stage)
        pltpu.sync_copy(stage, dsh.at[pl.ds(sid * RPS, RPS)])
        plsc.subcore_barrier()
        pltpu.sync_copy(ones_h, onesv)
        pltpu.sync_copy(dsti.at[pl.ds(wid * RPT, RPT)], dstv)

        def body(g, carry):
            r0 = g * _GK
            sd = [
                pltpu.async_copy(onesv, dsh.at[dstv.at[r0 + b]], sem, add=True)
                for b in range(_GK)
            ]
            for d in sd:
                d.wait()
            return carry

        lax.fori_loop(0, GRP, body, 0)
        plsc.subcore_barrier()
        pltpu.sync_copy(dsh.at[pl.ds(sid * RPS, RPS)], stage)
        pltpu.sync_copy(stage, out.at[pl.ds(cid * NP + sid * RPS, RPS)])

    return deg


def _tc1(degt, xp, W1, BM=1024):
    NP, DI = xp.shape
    DH = W1.shape[1]

    def body(deg_ref, x_ref, w_ref, o_ref):
        dg = deg_ref[:, 0:1] + deg_ref[:, 1:2] + 1.0
        dinv = lax.rsqrt(jnp.maximum(dg, 1e-12))
        h = jnp.dot(x_ref[...], w_ref[...], preferred_element_type=jnp.float32)
        o_ref[...] = h * dinv

    return pl.pallas_call(
        body,
        grid=(NP // BM,),
        in_specs=[
            pl.BlockSpec((BM, 2), lambda i: (i, 0)),
            pl.BlockSpec((BM, DI), lambda i: (i, 0)),
            pl.BlockSpec((DI, DH), lambda i: (0, 0)),
        ],
        out_specs=pl.BlockSpec((BM, DH), lambda i: (i, 0)),
        out_shape=jax.ShapeDtypeStruct((NP, DH), jnp.float32),
    )(degt, xp, W1)


def _tc2(degt, hs1, acc1, b1r, W2p, BM=1024):
    NP, DH = hs1.shape

    def body(deg_ref, hs_ref, acc_ref, b_ref, w_ref, o_ref):
        dg = deg_ref[:, 0:1] + deg_ref[:, 1:2] + 1.0
        dinv = lax.rsqrt(jnp.maximum(dg, 1e-12))
        s = hs_ref[...] + acc_ref[0] + acc_ref[1]
        g = jnp.maximum(dinv * s + b_ref[...], 0.0)
        o_ref[...] = jnp.dot(g, w_ref[...],
                             preferred_element_type=jnp.float32) * dinv

    return pl.pallas_call(
        body,
        grid=(NP // BM,),
        in_specs=[
            pl.BlockSpec((BM, 2), lambda i: (i, 0)),
            pl.BlockSpec((BM, DH), lambda i: (i, 0)),
            pl.BlockSpec((2, BM, DH), lambda i: (0, i, 0)),
            pl.BlockSpec((1, DH), lambda i: (0, 0)),
            pl.BlockSpec((DH, DH), lambda i: (0, 0)),
        ],
        out_specs=pl.BlockSpec((BM, DH), lambda i: (i, 0)),
        out_shape=jax.ShapeDtypeStruct((NP, DH), jnp.float32),
    )(degt, hs1, acc1, b1r, W2p)


def _tc3(degt, hs2, acc2, b2r, BM=1024):
    NP, DH = hs2.shape

    def body(deg_ref, hs_ref, acc_ref, b_ref, o_ref):
        dg = deg_ref[:, 0:1] + deg_ref[:, 1:2] + 1.0
        dinv = lax.rsqrt(jnp.maximum(dg, 1e-12))
        s = hs_ref[...] + acc_ref[0] + acc_ref[1]
        o_ref[...] = dinv * s + b_ref[...]

    return pl.pallas_call(
        body,
        grid=(NP // BM,),
        in_specs=[
            pl.BlockSpec((BM, 2), lambda i: (i, 0)),
            pl.BlockSpec((BM, DH), lambda i: (i, 0)),
            pl.BlockSpec((2, BM, DH), lambda i: (0, i, 0)),
            pl.BlockSpec((1, DH), lambda i: (0, 0)),
        ],
        out_specs=pl.BlockSpec((BM, DH), lambda i: (i, 0)),
        out_shape=jax.ShapeDtypeStruct((NP, DH), jnp.float32),
    )(degt, hs2, acc2, b2r)


def kernel(x, edge_index, W1, b1, W2, b2):
    N, DI = x.shape
    DH = W1.shape[1]
    DO = W2.shape[1]
    E = edge_index.shape[1]

    NP = -(-(N + 1) // 1024) * 1024
    RPT = -(-E // (_LANES * _W * _GK)) * _GK
    EP = _W * RPT * _LANES
    RPS = NP // _NS

    src = edge_index[0]
    dst = edge_index[1]
    pad = N + (jnp.arange(EP - E, dtype=jnp.int32) % (NP - N))
    srcr = jnp.concatenate([src, pad]).reshape(-1, _LANES)
    dstr = jnp.concatenate([dst, pad]).reshape(-1, _LANES)

    xp = jnp.pad(x, ((0, NP - N), (0, 0)))
    ones = jnp.ones((_LANES,), jnp.float32)
    zcol = jnp.zeros((RPS,), jnp.float32)
    zrows = jnp.zeros((RPS, DH), jnp.float32)
    W2p = jnp.pad(W2, ((0, 0), (0, DH - DO)))
    b1r = b1.reshape(1, DH)
    b2r = jnp.pad(b2, (0, DH - DO)).reshape(1, DH)

    deg2 = _make_deg(NP, RPT)(dstr, ones, zcol).reshape(_NC, NP)
    degt = deg2.T

    agg = _make_agg(NP, RPT, DH)

    hs1 = _tc1(degt, xp, W1)
    acc1 = agg(hs1, srcr, dstr, zrows).reshape(_NC, NP, DH)
    hs2 = _tc2(degt, hs1, acc1, b1r, W2p)
    acc2 = agg(hs2, srcr, dstr, zrows).reshape(_NC, NP, DH)
    outp = _tc3(degt, hs2, acc2, b2r)
    return outp[:N, :DO]
